# baseline (device time: 62358 ns/iter reference)
import jax
import jax.numpy as jnp
from jax import lax
from jax.experimental import pallas as pl
from jax.experimental.pallas import tpu as pltpu

CX = 4
CF = CX // 2


def kernel(O, Wo):
    B, S, H, D = O.shape
    HD = H * D
    N = Wo.shape[1]
    S_half = S // 2
    E = S_half // 4
    CE = E // CX
    MYCS = S_half // CX

    O2 = O.reshape(B, S, HD)

    def body(o_ref, w_ref, out_ref, res, xsend, xrecv, ydrecv, zdrecv,
             yfrecv, zfrecv, dma_s, xs_s, xr_s, yds_s, ydr_s, zds_s, zdr_s,
             yfs_s, yfr_s, zfs_s, zfr_s):
        my_x = lax.axis_index("x")
        my_y = lax.axis_index("y")
        my_z = lax.axis_index("z")
        xnbr = (1 - my_x, my_y, my_z)
        ynbr = (my_x, 1 - my_y, my_z)
        znbr = (my_x, my_y, 1 - my_z)

        j = 2 * my_y + my_z
        jy = 2 * (1 - my_y) + my_z
        jz = 2 * my_y + (1 - my_z)
        jd = 2 * (1 - my_y) + (1 - my_z)

        barrier = pltpu.get_barrier_semaphore()
        for nbr in (xnbr, ynbr, znbr):
            pl.semaphore_signal(
                barrier, inc=1, device_id=nbr,
                device_id_type=pl.DeviceIdType.MESH,
            )
        pl.semaphore_wait(barrier, 3)

        def copy(src, dst, ssem, rsem, nbr):
            return pltpu.make_async_remote_copy(
                src_ref=src, dst_ref=dst, send_sem=ssem, recv_sem=rsem,
                device_id=nbr, device_id_type=pl.DeviceIdType.MESH,
            )

        xs_start = (1 - my_x) * S_half + j * E
        x_rdmas = []
        for c in range(CX):
            for b in range(B):
                xsend[c, b] = jnp.dot(
                    o_ref[b, pl.ds(xs_start + c * CE, CE), :], w_ref[...],
                    preferred_element_type=jnp.float32,
                )
            r = copy(xsend.at[c], xrecv.at[c], xs_s.at[c], xr_s.at[c], xnbr)
            r.start()
            x_rdmas.append(r)

        my_start = my_x * S_half
        yd_rdmas, zd_rdmas = [], []
        for c in range(CX):
            x_rdmas[c].wait_recv()
            r = copy(xrecv.at[c], ydrecv.at[c], yds_s.at[c], ydr_s.at[c], ynbr)
            r.start()
            yd_rdmas.append(r)
            r = copy(xrecv.at[c], zdrecv.at[c], zds_s.at[c], zdr_s.at[c], znbr)
            r.start()
            zd_rdmas.append(r)
            for b in range(B):
                res[b, c * MYCS:(c + 1) * MYCS, :] = jnp.dot(
                    o_ref[b, pl.ds(my_start + c * MYCS, MYCS), :], w_ref[...],
                    preferred_element_type=jnp.float32,
                )

        yf_rdmas = []
        for c in range(CF):
            zd_rdmas[c].wait_recv()
            r = copy(zdrecv.at[c], yfrecv.at[c], yfs_s.at[c], yfr_s.at[c], ynbr)
            r.start()
            yf_rdmas.append(r)
        zf_rdmas = []
        for c in range(CF, CX):
            yd_rdmas[c].wait_recv()
            r = copy(ydrecv.at[c], zfrecv.at[c - CF], zfs_s.at[c - CF],
                     zfr_s.at[c - CF], znbr)
            r.start()
            zf_rdmas.append(r)

        pending = [None]

        def flush(q, first):
            d = pltpu.make_async_copy(
                res.at[:, pl.ds(q * E, E), :],
                out_ref.at[:, pl.ds(q * E, E), :], dma_s)
            if not first:
                pending[0].wait()
            d.start()
            pending[0] = d

        for c in range(CX):
            for b in range(B):
                res[b, pl.ds(j * E + c * CE, CE), :] += xrecv[c, b]
        flush(j, True)
        for c in range(CF, CX):
            zd_rdmas[c].wait_recv()
        for c in range(CF):
            yd_rdmas[c].wait_recv()
        for c in range(CX):
            for b in range(B):
                res[b, pl.ds(jy * E + c * CE, CE), :] += ydrecv[c, b]
        flush(jy, False)
        for c in range(CX):
            for b in range(B):
                res[b, pl.ds(jz * E + c * CE, CE), :] += zdrecv[c, b]
        flush(jz, False)
        for c in range(CF):
            yf_rdmas[c].wait_recv()
            for b in range(B):
                res[b, pl.ds(jd * E + c * CE, CE), :] += yfrecv[c, b]
        for c in range(CF):
            zf_rdmas[c].wait_recv()
            for b in range(B):
                res[b, pl.ds(jd * E + (CF + c) * CE, CE), :] += zfrecv[c, b]
        flush(jd, False)
        pending[0].wait()

        for r in x_rdmas + yd_rdmas + zd_rdmas + yf_rdmas + zf_rdmas:
            r.wait_send()

    chunk = (CX, B, CE, N)
    half = (CF, B, CE, N)
    return pl.pallas_call(
        body,
        out_shape=jax.ShapeDtypeStruct((B, S_half, N), jnp.float32),
        in_specs=[
            pl.BlockSpec(memory_space=pltpu.VMEM),
            pl.BlockSpec(memory_space=pltpu.VMEM),
        ],
        out_specs=pl.BlockSpec(memory_space=pltpu.MemorySpace.HBM),
        scratch_shapes=[
            pltpu.VMEM((B, S_half, N), jnp.float32),
            pltpu.VMEM(chunk, jnp.float32),
            pltpu.VMEM(chunk, jnp.float32),
            pltpu.VMEM(chunk, jnp.float32),
            pltpu.VMEM(chunk, jnp.float32),
            pltpu.VMEM(half, jnp.float32),
            pltpu.VMEM(half, jnp.float32),
            pltpu.SemaphoreType.DMA,
            pltpu.SemaphoreType.DMA((CX,)),
            pltpu.SemaphoreType.DMA((CX,)),
            pltpu.SemaphoreType.DMA((CX,)),
            pltpu.SemaphoreType.DMA((CX,)),
            pltpu.SemaphoreType.DMA((CX,)),
            pltpu.SemaphoreType.DMA((CX,)),
            pltpu.SemaphoreType.DMA((CF,)),
            pltpu.SemaphoreType.DMA((CF,)),
            pltpu.SemaphoreType.DMA((CF,)),
            pltpu.SemaphoreType.DMA((CF,)),
        ],
        compiler_params=pltpu.CompilerParams(collective_id=0),
    )(O2, Wo)


# device time: 62190 ns/iter; 1.0027x vs baseline; 1.0027x over previous
import jax
import jax.numpy as jnp
from jax import lax
from jax.experimental import pallas as pl
from jax.experimental.pallas import tpu as pltpu

CX = 4
CF = CX // 2


def kernel(O, Wo):
    B, S, H, D = O.shape
    HD = H * D
    N = Wo.shape[1]
    S_half = S // 2
    E = S_half // 4
    CE = E // CX
    MYCS = S_half // CX

    O2 = O.reshape(B, S, HD)

    def body(o_ref, w_ref, out_ref, xsend, xrecv, ydrecv, zdrecv,
             yfrecv, zfrecv, xs_s, xr_s, yds_s, ydr_s, zds_s, zdr_s,
             yfs_s, yfr_s, zfs_s, zfr_s):
        my_x = lax.axis_index("x")
        my_y = lax.axis_index("y")
        my_z = lax.axis_index("z")
        xnbr = (1 - my_x, my_y, my_z)
        ynbr = (my_x, 1 - my_y, my_z)
        znbr = (my_x, my_y, 1 - my_z)

        j = 2 * my_y + my_z
        jy = 2 * (1 - my_y) + my_z
        jz = 2 * my_y + (1 - my_z)
        jd = 2 * (1 - my_y) + (1 - my_z)

        barrier = pltpu.get_barrier_semaphore()
        for nbr in (xnbr, ynbr, znbr):
            pl.semaphore_signal(
                barrier, inc=1, device_id=nbr,
                device_id_type=pl.DeviceIdType.MESH,
            )
        pl.semaphore_wait(barrier, 3)

        def copy(src, dst, ssem, rsem, nbr):
            return pltpu.make_async_remote_copy(
                src_ref=src, dst_ref=dst, send_sem=ssem, recv_sem=rsem,
                device_id=nbr, device_id_type=pl.DeviceIdType.MESH,
            )

        xs_start = (1 - my_x) * S_half + j * E
        x_rdmas = []
        for c in range(CX):
            for b in range(B):
                xsend[c, b] = jnp.dot(
                    o_ref[b, pl.ds(xs_start + c * CE, CE), :], w_ref[...],
                    preferred_element_type=jnp.float32,
                )
            r = copy(xsend.at[c], xrecv.at[c], xs_s.at[c], xr_s.at[c], xnbr)
            r.start()
            x_rdmas.append(r)

        my_start = my_x * S_half
        yd_rdmas, zd_rdmas = [], []
        for c in range(CX):
            x_rdmas[c].wait_recv()
            r = copy(xrecv.at[c], ydrecv.at[c], yds_s.at[c], ydr_s.at[c], ynbr)
            r.start()
            yd_rdmas.append(r)
            r = copy(xrecv.at[c], zdrecv.at[c], zds_s.at[c], zdr_s.at[c], znbr)
            r.start()
            zd_rdmas.append(r)
            for b in range(B):
                out_ref[b, c * MYCS:(c + 1) * MYCS, :] = jnp.dot(
                    o_ref[b, pl.ds(my_start + c * MYCS, MYCS), :], w_ref[...],
                    preferred_element_type=jnp.float32,
                )

        yf_rdmas = []
        for c in range(CF):
            zd_rdmas[c].wait_recv()
            r = copy(zdrecv.at[c], yfrecv.at[c], yfs_s.at[c], yfr_s.at[c], ynbr)
            r.start()
            yf_rdmas.append(r)
        zf_rdmas = []
        for c in range(CF, CX):
            yd_rdmas[c].wait_recv()
            r = copy(ydrecv.at[c], zfrecv.at[c - CF], zfs_s.at[c - CF],
                     zfr_s.at[c - CF], znbr)
            r.start()
            zf_rdmas.append(r)

        for c in range(CX):
            for b in range(B):
                out_ref[b, pl.ds(j * E + c * CE, CE), :] += xrecv[c, b]
        for c in range(CF, CX):
            zd_rdmas[c].wait_recv()
        for c in range(CF):
            yd_rdmas[c].wait_recv()
        for c in range(CX):
            for b in range(B):
                out_ref[b, pl.ds(jy * E + c * CE, CE), :] += ydrecv[c, b]
        for c in range(CX):
            for b in range(B):
                out_ref[b, pl.ds(jz * E + c * CE, CE), :] += zdrecv[c, b]
        for c in range(CF):
            yf_rdmas[c].wait_recv()
            for b in range(B):
                out_ref[b, pl.ds(jd * E + c * CE, CE), :] += yfrecv[c, b]
        for c in range(CF):
            zf_rdmas[c].wait_recv()
            for b in range(B):
                out_ref[b, pl.ds(jd * E + (CF + c) * CE, CE), :] += zfrecv[c, b]

        for r in x_rdmas + yd_rdmas + zd_rdmas + yf_rdmas + zf_rdmas:
            r.wait_send()

    chunk = (CX, B, CE, N)
    half = (CF, B, CE, N)
    return pl.pallas_call(
        body,
        out_shape=jax.ShapeDtypeStruct((B, S_half, N), jnp.float32),
        in_specs=[
            pl.BlockSpec(memory_space=pltpu.VMEM),
            pl.BlockSpec(memory_space=pltpu.VMEM),
        ],
        out_specs=pl.BlockSpec(memory_space=pltpu.VMEM),
        scratch_shapes=[
            pltpu.VMEM(chunk, jnp.float32),
            pltpu.VMEM(chunk, jnp.float32),
            pltpu.VMEM(chunk, jnp.float32),
            pltpu.VMEM(chunk, jnp.float32),
            pltpu.VMEM(half, jnp.float32),
            pltpu.VMEM(half, jnp.float32),
            pltpu.SemaphoreType.DMA((CX,)),
            pltpu.SemaphoreType.DMA((CX,)),
            pltpu.SemaphoreType.DMA((CX,)),
            pltpu.SemaphoreType.DMA((CX,)),
            pltpu.SemaphoreType.DMA((CX,)),
            pltpu.SemaphoreType.DMA((CX,)),
            pltpu.SemaphoreType.DMA((CF,)),
            pltpu.SemaphoreType.DMA((CF,)),
            pltpu.SemaphoreType.DMA((CF,)),
            pltpu.SemaphoreType.DMA((CF,)),
        ],
        compiler_params=pltpu.CompilerParams(collective_id=0),
    )(O2, Wo)
